# baseline (speedup 1.0000x reference)
import jax
import jax.numpy as jnp
from jax import lax
from jax.experimental import pallas as pl
from jax.experimental.pallas import tpu as pltpu

N_TOK = 2048
D = 768
E = 64
H = 128
EB = 8
NEB = E // EB
WH = EB * H
NEG_BIG = -1e30
HI = jax.lax.Precision.HIGHEST


def _moe_body(x_ref, esel_ref, w1_ref, w2_ref, out_ref,
              xbf_ref, g0_ref, g1_ref, e0_ref, e1_ref):
    s = pl.program_id(0)

    @pl.when(s == 0)
    def _router():
        xbf = x_ref[...].astype(jnp.bfloat16)
        xbf_ref[...] = xbf
        scores = lax.dot_general(
            xbf, esel_ref[...].astype(jnp.bfloat16), (((1,), (1,)), ((), ())),
            preferred_element_type=jnp.float32)
        sel = jax.nn.sigmoid(scores)
        iota_e = lax.broadcasted_iota(jnp.int32, (N_TOK, E), 1
                                      ).astype(jnp.float32)
        m1 = jnp.max(sel, axis=1, keepdims=True)
        i1 = jnp.min(jnp.where(sel == m1, iota_e, float(E)), axis=1,
                     keepdims=True)
        sel2 = jnp.where(iota_e == i1, NEG_BIG, sel)
        m2 = jnp.max(sel2, axis=1, keepdims=True)
        i2 = jnp.min(jnp.where(sel2 == m2, iota_e, float(E)), axis=1,
                     keepdims=True)
        g0_ref[...] = m1
        g1_ref[...] = m2
        e0_ref[...] = i1
        e1_ref[...] = i2

    @pl.when(s > 0)
    def _experts():
        b = s - 1
        ids = lax.broadcasted_iota(jnp.int32, (N_TOK, EB), 1
                                   ).astype(jnp.float32) + (b * EB)
        c8 = (g0_ref[...] * (e0_ref[...] == ids)
              + g1_ref[...] * (e1_ref[...] == ids))
        rr = lax.broadcasted_iota(jnp.int32, (EB, WH), 0)
        ll = lax.broadcasted_iota(jnp.int32, (EB, WH), 1)
        b8 = ((ll >> 7) == rr).astype(jnp.float32)
        cmat = lax.dot_general(c8, b8, (((1,), (0,)), ((), ())),
                               precision=HI,
                               preferred_element_type=jnp.float32)
        xbf = xbf_ref[...]
        h = jnp.concatenate(
            [lax.dot_general(xbf, w1_ref[i].astype(jnp.bfloat16),
                             (((1,), (0,)), ((), ())),
                             preferred_element_type=jnp.float32)
             for i in range(EB)], axis=1)
        hw = (jnp.maximum(h, 0.0) * cmat).astype(jnp.bfloat16)
        w2cat = w2_ref[...].astype(jnp.bfloat16).reshape(WH, D)
        o = lax.dot_general(hw, w2cat, (((1,), (0,)), ((), ())),
                            preferred_element_type=jnp.float32)

        @pl.when(s == 1)
        def _init():
            out_ref[...] = o

        @pl.when(s > 1)
        def _acc():
            out_ref[...] += o


@jax.jit
def kernel(x, expert_sel, W1, W2):
    we_idx = lambda s: (jnp.maximum(s - 1, 0), 0, 0)
    return pl.pallas_call(
        _moe_body,
        grid=(NEB + 1,),
        in_specs=[
            pl.BlockSpec((N_TOK, D), lambda s: (0, 0)),
            pl.BlockSpec((E, D), lambda s: (0, 0)),
            pl.BlockSpec((EB, D, H), we_idx),
            pl.BlockSpec((EB, H, D), we_idx),
        ],
        out_specs=pl.BlockSpec((N_TOK, D), lambda s: (0, 0)),
        out_shape=jax.ShapeDtypeStruct((N_TOK, D), jnp.float32),
        scratch_shapes=[
            pltpu.VMEM((N_TOK, D), jnp.bfloat16),
            pltpu.VMEM((N_TOK, 1), jnp.float32),
            pltpu.VMEM((N_TOK, 1), jnp.float32),
            pltpu.VMEM((N_TOK, 1), jnp.float32),
            pltpu.VMEM((N_TOK, 1), jnp.float32),
        ],
    )(x, expert_sel, W1, W2)


# speedup vs baseline: 1.9906x; 1.9906x over previous
import jax
import jax.numpy as jnp
from jax import lax
from jax.experimental import pallas as pl
from jax.experimental.pallas import tpu as pltpu

N_TOK = 2048
D = 768
E = 64
H = 128
EB = 8
NEB = E // EB
WH = EB * H
NEG_BIG = -1e30
HI = jax.lax.Precision.HIGHEST


def _moe_body(x_ref, esel_ref, w1_ref, w2_ref, out_ref,
              xbf_ref, g0_ref, g1_ref, e0_ref, e1_ref):
    s = pl.program_id(0)

    @pl.when(s == 0)
    def _router():
        xbf = x_ref[...].astype(jnp.bfloat16)
        xbf_ref[...] = xbf
        scores = lax.dot_general(
            xbf, esel_ref[...].astype(jnp.bfloat16), (((1,), (1,)), ((), ())),
            preferred_element_type=jnp.float32)
        sel = jax.nn.sigmoid(scores)
        iota_e = lax.broadcasted_iota(jnp.int32, (N_TOK, E), 1
                                      ).astype(jnp.float32)
        m1 = jnp.max(sel, axis=1, keepdims=True)
        i1 = jnp.min(jnp.where(sel == m1, iota_e, float(E)), axis=1,
                     keepdims=True)
        sel2 = jnp.where(iota_e == i1, NEG_BIG, sel)
        m2 = jnp.max(sel2, axis=1, keepdims=True)
        i2 = jnp.min(jnp.where(sel2 == m2, iota_e, float(E)), axis=1,
                     keepdims=True)
        g0_ref[...] = m1
        g1_ref[...] = m2
        e0_ref[...] = i1
        e1_ref[...] = i2

    @pl.when(s > 0)
    def _experts():
        b = s - 1
        ids = (lax.broadcasted_iota(jnp.int32, (N_TOK, WH), 1) >> 7
               ).astype(jnp.float32) + (b * EB)
        xbf = xbf_ref[...]
        w1cat = jnp.transpose(w1_ref[...].astype(jnp.bfloat16),
                              (1, 0, 2)).reshape(D, WH)
        h = lax.dot_general(xbf, w1cat, (((1,), (0,)), ((), ())),
                            preferred_element_type=jnp.float32)
        cmat = jnp.where(e0_ref[...] == ids, g0_ref[...],
                         jnp.where(e1_ref[...] == ids, g1_ref[...], 0.0))
        hw = (jnp.maximum(h, 0.0) * cmat).astype(jnp.bfloat16)
        w2cat = w2_ref[...].astype(jnp.bfloat16).reshape(WH, D)
        o = lax.dot_general(hw, w2cat, (((1,), (0,)), ((), ())),
                            preferred_element_type=jnp.float32)

        @pl.when(s == 1)
        def _init():
            out_ref[...] = o

        @pl.when(s > 1)
        def _acc():
            out_ref[...] += o


@jax.jit
def kernel(x, expert_sel, W1, W2):
    we_idx = lambda s: (jnp.maximum(s - 1, 0), 0, 0)
    return pl.pallas_call(
        _moe_body,
        grid=(NEB + 1,),
        in_specs=[
            pl.BlockSpec((N_TOK, D), lambda s: (0, 0)),
            pl.BlockSpec((E, D), lambda s: (0, 0)),
            pl.BlockSpec((EB, D, H), we_idx),
            pl.BlockSpec((EB, H, D), we_idx),
        ],
        out_specs=pl.BlockSpec((N_TOK, D), lambda s: (0, 0)),
        out_shape=jax.ShapeDtypeStruct((N_TOK, D), jnp.float32),
        scratch_shapes=[
            pltpu.VMEM((N_TOK, D), jnp.bfloat16),
            pltpu.VMEM((N_TOK, 1), jnp.float32),
            pltpu.VMEM((N_TOK, 1), jnp.float32),
            pltpu.VMEM((N_TOK, 1), jnp.float32),
            pltpu.VMEM((N_TOK, 1), jnp.float32),
        ],
    )(x, expert_sel, W1, W2)


# speedup vs baseline: 2.1122x; 1.0611x over previous
import jax
import jax.numpy as jnp
from jax import lax
from jax.experimental import pallas as pl
from jax.experimental.pallas import tpu as pltpu

N_TOK = 2048
D = 768
E = 64
H = 128
EB = 8
NEB = E // EB
WH = EB * H
NEG_BIG = -1e30
HI = jax.lax.Precision.HIGHEST


def _moe_body(x_ref, esel_ref, w1_ref, w2_ref, out_ref,
              xbf_ref, g0_ref, g1_ref, e0_ref, e1_ref):
    s = pl.program_id(0)

    @pl.when(s == 0)
    def _router():
        xbf = x_ref[...].astype(jnp.bfloat16)
        xbf_ref[...] = xbf
        scores = lax.dot_general(
            xbf, esel_ref[...].astype(jnp.bfloat16), (((1,), (1,)), ((), ())),
            preferred_element_type=jnp.float32)
        sel = jax.nn.sigmoid(scores)
        iota_e = lax.broadcasted_iota(jnp.int32, (N_TOK, E), 1
                                      ).astype(jnp.float32)
        m1 = jnp.max(sel, axis=1, keepdims=True)
        i1 = jnp.min(jnp.where(sel == m1, iota_e, float(E)), axis=1,
                     keepdims=True)
        sel2 = jnp.where(iota_e == i1, NEG_BIG, sel)
        m2 = jnp.max(sel2, axis=1, keepdims=True)
        i2 = jnp.min(jnp.where(sel2 == m2, iota_e, float(E)), axis=1,
                     keepdims=True)
        g0_ref[...] = m1
        g1_ref[...] = m2
        e0_ref[...] = i1
        e1_ref[...] = i2

    @pl.when(s > 0)
    def _experts():
        b = s - 1
        ids = (lax.broadcasted_iota(jnp.int32, (N_TOK, WH), 1) >> 7
               ).astype(jnp.float32) + (b * EB)
        xbf = xbf_ref[...]
        w1cat = jnp.transpose(w1_ref[...].astype(jnp.bfloat16),
                              (1, 0, 2)).reshape(D, WH)
        h = lax.dot_general(xbf, w1cat, (((1,), (0,)), ((), ())),
                            preferred_element_type=jnp.float32)
        cmat = jnp.where(e0_ref[...] == ids, g0_ref[...],
                         jnp.where(e1_ref[...] == ids, g1_ref[...], 0.0))
        hw = (jnp.maximum(h, 0.0) * cmat).astype(jnp.bfloat16)
        w2cat = w2_ref[...].astype(jnp.bfloat16).reshape(WH, D)
        o = lax.dot_general(hw, w2cat, (((1,), (0,)), ((), ())),
                            preferred_element_type=jnp.float32)
        out_ref[...] = jnp.where(s > 1, out_ref[...], 0.0) + o


@jax.jit
def kernel(x, expert_sel, W1, W2):
    we_idx = lambda s: (jnp.maximum(s - 1, 0), 0, 0)
    return pl.pallas_call(
        _moe_body,
        grid=(NEB + 1,),
        in_specs=[
            pl.BlockSpec((N_TOK, D), lambda s: (0, 0)),
            pl.BlockSpec((E, D), lambda s: (0, 0)),
            pl.BlockSpec((EB, D, H), we_idx),
            pl.BlockSpec((EB, H, D), we_idx),
        ],
        out_specs=pl.BlockSpec((N_TOK, D), lambda s: (0, 0)),
        out_shape=jax.ShapeDtypeStruct((N_TOK, D), jnp.float32),
        scratch_shapes=[
            pltpu.VMEM((N_TOK, D), jnp.bfloat16),
            pltpu.VMEM((N_TOK, 1), jnp.float32),
            pltpu.VMEM((N_TOK, 1), jnp.float32),
            pltpu.VMEM((N_TOK, 1), jnp.float32),
            pltpu.VMEM((N_TOK, 1), jnp.float32),
        ],
    )(x, expert_sel, W1, W2)


# speedup vs baseline: 2.1706x; 1.0277x over previous
import jax
import jax.numpy as jnp
from jax import lax
from jax.experimental import pallas as pl
from jax.experimental.pallas import tpu as pltpu

N_TOK = 2048
D = 768
E = 64
H = 128
EB = 8
NEB = E // EB
WH = EB * H
NEG_BIG = -1e30
HI = jax.lax.Precision.HIGHEST


def _moe_body(x_ref, esel_ref, w1_ref, w2_ref, out_ref,
              xbf_ref, g0_ref, g1_ref, e0_ref, e1_ref):
    s = pl.program_id(0)

    @pl.when(s == 0)
    def _router():
        xbf = x_ref[...].astype(jnp.bfloat16)
        xbf_ref[...] = xbf
        scores = lax.dot_general(
            xbf, esel_ref[...].astype(jnp.bfloat16), (((1,), (1,)), ((), ())),
            preferred_element_type=jnp.float32)
        sel = jax.nn.sigmoid(scores)
        iota_e = lax.broadcasted_iota(jnp.int32, (N_TOK, E), 1
                                      ).astype(jnp.float32)
        m1 = jnp.max(sel, axis=1, keepdims=True)
        i1 = jnp.min(jnp.where(sel == m1, iota_e, float(E)), axis=1,
                     keepdims=True)
        sel2 = jnp.where(iota_e == i1, NEG_BIG, sel)
        m2 = jnp.max(sel2, axis=1, keepdims=True)
        i2 = jnp.min(jnp.where(sel2 == m2, iota_e, float(E)), axis=1,
                     keepdims=True)
        g0_ref[...] = m1
        g1_ref[...] = m2
        e0_ref[...] = i1
        e1_ref[...] = i2

    @pl.when(s > 0)
    def _experts():
        b = s - 1
        ids = (lax.broadcasted_iota(jnp.int32, (N_TOK, WH), 1) >> 7
               ).astype(jnp.float32) + (b * EB)
        xbf = xbf_ref[...]
        w1cat = jnp.concatenate(
            [w1_ref[i].astype(jnp.bfloat16) for i in range(EB)], axis=1)
        h = lax.dot_general(xbf, w1cat, (((1,), (0,)), ((), ())),
                            preferred_element_type=jnp.float32)
        cmat = jnp.where(e0_ref[...] == ids, g0_ref[...],
                         jnp.where(e1_ref[...] == ids, g1_ref[...], 0.0))
        hw = (jnp.maximum(h, 0.0) * cmat).astype(jnp.bfloat16)
        w2cat = w2_ref[...].astype(jnp.bfloat16).reshape(WH, D)
        o = lax.dot_general(hw, w2cat, (((1,), (0,)), ((), ())),
                            preferred_element_type=jnp.float32)
        out_ref[...] = jnp.where(s > 1, out_ref[...], 0.0) + o


@jax.jit
def kernel(x, expert_sel, W1, W2):
    we_idx = lambda s: (jnp.maximum(s - 1, 0), 0, 0)
    return pl.pallas_call(
        _moe_body,
        grid=(NEB + 1,),
        in_specs=[
            pl.BlockSpec((N_TOK, D), lambda s: (0, 0)),
            pl.BlockSpec((E, D), lambda s: (0, 0)),
            pl.BlockSpec((EB, D, H), we_idx),
            pl.BlockSpec((EB, H, D), we_idx),
        ],
        out_specs=pl.BlockSpec((N_TOK, D), lambda s: (0, 0)),
        out_shape=jax.ShapeDtypeStruct((N_TOK, D), jnp.float32),
        scratch_shapes=[
            pltpu.VMEM((N_TOK, D), jnp.bfloat16),
            pltpu.VMEM((N_TOK, 1), jnp.float32),
            pltpu.VMEM((N_TOK, 1), jnp.float32),
            pltpu.VMEM((N_TOK, 1), jnp.float32),
            pltpu.VMEM((N_TOK, 1), jnp.float32),
        ],
    )(x, expert_sel, W1, W2)
